# baseline (speedup 1.0000x reference)
import functools

import jax
import jax.numpy as jnp
from jax import lax
from jax.experimental import pallas as pl
from jax.experimental.pallas import tpu as pltpu
from jax.experimental.pallas import tpu_sc as plsc

_B = 16
_A = 8525
_NC = 80
_TA = 8576
_OFF = (0, 6400, 8000, 8400, 8500, 8525)
_EPS = 1e-7
_HI = jax.lax.Precision.HIGHEST

_N = _B * _A
_NA = 4264
_NL = _N - 31 * _NA
_NS = 4272
_ITERS = _NS // 16


def _perm16(vals):
    def g(v, perm):
        return lax.gather(
            v, perm[:, None],
            dimension_numbers=lax.GatherDimensionNumbers(
                offset_dims=(), collapsed_slice_dims=(0,),
                start_index_map=(0,)),
            slice_sizes=(1,),
            mode=lax.GatherScatterMode.PROMISE_IN_BOUNDS)
    return g(*vals)


@functools.partial(
    pl.kernel,
    out_type=jax.ShapeDtypeStruct((_N,), jnp.float32),
    mesh=plsc.VectorSubcoreMesh(core_axis_name="c", subcore_axis_name="s"),
    scratch_types=[
        pltpu.VMEM((_NS * 4,), jnp.float32),
        pltpu.VMEM((_NS * 4,), jnp.float32),
        pltpu.VMEM((_NS,), jnp.float32),
    ],
)
def _giou_sc(lp_hbm, lt_hbm, out_hbm, lp_v, lt_v, o_v):
    wid = lax.axis_index("s") * 2 + lax.axis_index("c")
    base = wid * _NA

    @pl.when(wid < 31)
    def _():
        pltpu.sync_copy(lp_hbm.at[pl.ds(base * 4, _NA * 4)], lp_v.at[pl.ds(0, _NA * 4)])
        pltpu.sync_copy(lt_hbm.at[pl.ds(base * 4, _NA * 4)], lt_v.at[pl.ds(0, _NA * 4)])

    @pl.when(wid == 31)
    def _():
        pltpu.sync_copy(lp_hbm.at[pl.ds(base * 4, _NL * 4)], lp_v.at[pl.ds(0, _NL * 4)])
        pltpu.sync_copy(lt_hbm.at[pl.ds(base * 4, _NL * 4)], lt_v.at[pl.ds(0, _NL * 4)])

    lanes = lax.iota(jnp.int32, 16)
    in0 = lanes < 4
    in1 = lanes < 8
    in2 = lanes < 12
    perms = [(lanes % 4) * 4 + c for c in range(4)]

    def deint(ref, j, c):
        vs = [ref[pl.ds(j * 64 + q * 16, 16)] for q in range(4)]
        ps = [_perm16((v, perms[c])) for v in vs]
        return jnp.where(in0, ps[0],
                         jnp.where(in1, ps[1],
                                   jnp.where(in2, ps[2], ps[3])))

    def body(j, carry):
        tx1 = deint(lt_v, j, 0)
        ty1 = deint(lt_v, j, 1)
        tx2 = deint(lt_v, j, 2)
        ty2 = deint(lt_v, j, 3)
        px1 = deint(lp_v, j, 0)
        py1 = deint(lp_v, j, 1)
        px2 = deint(lp_v, j, 2)
        py2 = deint(lp_v, j, 3)
        t1x = jnp.minimum(tx1, tx2)
        t2x = jnp.maximum(tx1, tx2)
        t1y = jnp.minimum(ty1, ty2)
        t2y = jnp.maximum(ty1, ty2)
        p1x = jnp.minimum(px1, px2)
        p2x = jnp.maximum(px1, px2)
        p1y = jnp.minimum(py1, py2)
        p2y = jnp.maximum(py1, py2)
        iw = jnp.maximum(jnp.minimum(t2x, p2x) - jnp.maximum(t1x, p1x), 0.0)
        ih = jnp.maximum(jnp.minimum(t2y, p2y) - jnp.maximum(t1y, p1y), 0.0)
        inter = iw * ih
        area_t = (t2x - t1x) * (t2y - t1y)
        area_p = (p2x - p1x) * (p2y - p1y)
        union = area_t + area_p - inter
        iou = inter / (union + _EPS)
        ew = jnp.maximum(t2x, p2x) - jnp.minimum(t1x, p1x)
        eh = jnp.maximum(t2y, p2y) - jnp.minimum(t1y, p1y)
        enc = ew * eh
        giou = iou - (enc - union) / (enc + _EPS)
        o_v[pl.ds(j * 16, 16)] = 1.0 - giou
        return carry

    lax.fori_loop(0, _ITERS, body, 0)

    @pl.when(wid < 31)
    def _():
        pltpu.sync_copy(o_v.at[pl.ds(0, _NA)], out_hbm.at[pl.ds(base, _NA)])

    @pl.when(wid == 31)
    def _():
        pltpu.sync_copy(o_v.at[pl.ds(0, _NL)], out_hbm.at[pl.ds(base, _NL)])


def _lane_masks():
    ga = jax.lax.broadcasted_iota(jnp.int32, (1, _TA), 1)
    validT = ga < _A
    lmasks = [
        jnp.where((ga >= _OFF[l]) & (ga < _OFF[l + 1]), 1.0, 0.0)
        for l in range(5)
    ]
    return validT, lmasks


def _focal_kernel(cls_pred_ref, cls_tar_ref, closs_ref, msk_ref):
    x = cls_pred_ref[0]
    y = cls_tar_ref[0][:, :_NC]
    e = jnp.exp(-x)
    ce = jnp.log(1.0 + e) + x - y * x
    p = 1.0 / (1.0 + e)
    om = p + y - 2.0 * y * p
    a_t = 0.75 - 0.5 * y
    f = a_t * (om * om) * ce
    ones = jnp.ones((1, _NC), jnp.float32)
    closs_ref[0] = jax.lax.dot_general(
        ones, f, (((1,), (1,)), ((), ())),
        preferred_element_type=jnp.float32, precision=_HI)
    sel = (jax.lax.broadcasted_iota(jnp.int32, (1, _NC + 2), 1)
           == _NC + 1).astype(jnp.float32)
    msk_ref[0] = jax.lax.dot_general(
        sel, cls_tar_ref[0], (((1,), (1,)), ((), ())),
        preferred_element_type=jnp.float32, precision=_HI)


def _combine_kernel(closs_ref, gl_ref, ind_ref, msk_ref, cnt_ref, out_ref):
    validT, lmasks = _lane_masks()
    total = jnp.where(validT, closs_ref[0] + gl_ref[0], 0.0)
    oid_iota = jax.lax.broadcasted_iota(jnp.int32, (8, 1), 0)
    oidoh = (ind_ref[0] == oid_iota).astype(jnp.float32)
    z = oidoh * total
    sums = jnp.concatenate(
        [jnp.sum(z * lm, axis=1, keepdims=True) for lm in lmasks], axis=1)
    cnts = jnp.concatenate(
        [jnp.sum(oidoh * lm, axis=1, keepdims=True) for lm in lmasks], axis=1)

    mean = sums / jnp.maximum(1.0, cnts)
    mn = jnp.min(mean, axis=1, keepdims=True)
    mx = jnp.max(mean, axis=1, keepdims=True)
    t = 1.0 - (mean - mn) / (mx - mn)
    m1 = jnp.min(t, axis=1, keepdims=True)
    eqm = t == m1
    nmin = jnp.sum(eqm.astype(jnp.float32), axis=1, keepdims=True)
    m2 = jnp.min(jnp.where(eqm, jnp.inf, t), axis=1, keepdims=True)
    mw = jnp.where(nmin >= 2.0, m1, m2)
    tk = jnp.where(t > mw, t, 0.0)
    cnt = cnt_ref[0, 0, 0]
    rowmask = jax.lax.broadcasted_iota(jnp.int32, (8, 5), 0) < cnt
    tk = jnp.where(rowmask, tk, 0.0)

    lvloh = jnp.concatenate(lmasks, axis=0)
    w = jax.lax.dot_general(tk, lvloh, (((1,), (0,)), ((), ())),
                            preferred_element_type=jnp.float32,
                            precision=_HI)
    gathered = jnp.sum(oidoh * w, axis=0, keepdims=True)
    out_ref[0] = jnp.where(msk_ref[0] > 0.0, gathered, 1.0)


@functools.partial(jax.jit, static_argnums=())
def kernel(cls_pred, loc_pred, cls_tar, loc_tar, ind_tar, bboxes_cnt):
    indT = ind_tar.astype(jnp.int32).reshape(_B, 1, _A)
    cnt32 = bboxes_cnt.astype(jnp.int32).reshape(_B, 1, 1)

    gl = _giou_sc(loc_pred.reshape(_N * 4),
                  loc_tar.reshape(_N * 4)).reshape(_B, 1, _A)

    closs, mskv = pl.pallas_call(
        _focal_kernel,
        grid=(_B,),
        in_specs=[
            pl.BlockSpec((1, _TA, _NC), lambda b: (b, 0, 0)),
            pl.BlockSpec((1, _TA, _NC + 2), lambda b: (b, 0, 0)),
        ],
        out_specs=[
            pl.BlockSpec((1, 1, _TA), lambda b: (b, 0, 0)),
            pl.BlockSpec((1, 1, _TA), lambda b: (b, 0, 0)),
        ],
        out_shape=[
            jax.ShapeDtypeStruct((_B, 1, _A), jnp.float32),
            jax.ShapeDtypeStruct((_B, 1, _A), jnp.float32),
        ],
    )(cls_pred, cls_tar)

    outT = pl.pallas_call(
        _combine_kernel,
        grid=(_B,),
        in_specs=[
            pl.BlockSpec((1, 1, _TA), lambda b: (b, 0, 0)),
            pl.BlockSpec((1, 1, _TA), lambda b: (b, 0, 0)),
            pl.BlockSpec((1, 1, _TA), lambda b: (b, 0, 0)),
            pl.BlockSpec((1, 1, _TA), lambda b: (b, 0, 0)),
            pl.BlockSpec((1, 1, 1), lambda b: (b, 0, 0),
                         memory_space=pltpu.SMEM),
        ],
        out_specs=pl.BlockSpec((1, 1, _TA), lambda b: (b, 0, 0)),
        out_shape=jax.ShapeDtypeStruct((_B, 1, _A), jnp.float32),
    )(closs, gl, indT, mskv, cnt32)

    return outT.reshape(_B, _A, 1)


# speedup vs baseline: 1.7702x; 1.7702x over previous
import functools

import jax
import jax.numpy as jnp
from jax.experimental import pallas as pl
from jax.experimental.pallas import tpu as pltpu

_B = 16
_A = 8525
_NC = 80
_TA = 8576
_C = 1
_OFF = (0, 6400, 8000, 8400, 8500, 8525)
_EPS = 1e-7
_HI = jax.lax.Precision.HIGHEST


def _lane_masks(c):
    ga = jax.lax.broadcasted_iota(jnp.int32, (1, _TA), 1) + c * _TA
    validT = ga < _A
    lmasks = [
        jnp.where((ga >= _OFF[l]) & (ga < _OFF[l + 1]), 1.0, 0.0)
        for l in range(5)
    ]
    return validT, lmasks


def _oid_onehot(ind_row):
    oid_iota = jax.lax.broadcasted_iota(jnp.int32, (8, 1), 0)
    return (ind_row == oid_iota).astype(jnp.float32)


def _stats_kernel(cls_pred_ref, cls_tar_ref, loc_pred_ref, loc_tar_ref,
                  ind_ref, cnt_ref, tstat_ref, msk_ref, acc_s, acc_c):
    c = pl.program_id(1)

    x = cls_pred_ref[0]
    y = cls_tar_ref[0][:, :_NC]
    e = jnp.exp(-x)
    ce = jnp.log(1.0 + e) + x - y * x
    p = 1.0 / (1.0 + e)
    om = p + y - 2.0 * y * p
    a_t = 0.75 - 0.5 * y
    f = a_t * (om * om) * ce
    ones = jnp.ones((1, _NC), jnp.float32)
    cls_loss = jax.lax.dot_general(
        ones, f, (((1,), (1,)), ((), ())),
        preferred_element_type=jnp.float32, precision=_HI)

    bt = loc_tar_ref[0]
    bp = loc_pred_ref[0]
    t1 = jnp.minimum(bt[0:2, :], bt[2:4, :])
    t2 = jnp.maximum(bt[0:2, :], bt[2:4, :])
    p1 = jnp.minimum(bp[0:2, :], bp[2:4, :])
    p2 = jnp.maximum(bp[0:2, :], bp[2:4, :])
    ihw = jnp.maximum(jnp.minimum(t2, p2) - jnp.maximum(t1, p1), 0.0)
    inter = ihw[0:1, :] * ihw[1:2, :]
    wht = t2 - t1
    whp = p2 - p1
    area_t = wht[0:1, :] * wht[1:2, :]
    area_p = whp[0:1, :] * whp[1:2, :]
    union = area_t + area_p - inter
    iou = inter / (union + _EPS)
    ewh = jnp.maximum(t2, p2) - jnp.minimum(t1, p1)
    enc = ewh[0:1, :] * ewh[1:2, :]
    giou = iou - (enc - union) / (enc + _EPS)
    loc_loss = 1.0 - giou

    validT, lmasks = _lane_masks(c)
    total = jnp.where(validT, cls_loss + loc_loss, 0.0)
    oidoh = _oid_onehot(ind_ref[0])
    z = oidoh * total
    sums = jnp.concatenate(
        [jnp.sum(z * lm, axis=1, keepdims=True) for lm in lmasks], axis=1)
    cnts = jnp.concatenate(
        [jnp.sum(oidoh * lm, axis=1, keepdims=True) for lm in lmasks], axis=1)

    @pl.when(c == 0)
    def _():
        acc_s[:, 0:5] = sums
        acc_c[:, 0:5] = cnts

    @pl.when(c > 0)
    def _():
        acc_s[:, 0:5] += sums
        acc_c[:, 0:5] += cnts

    sel = (jax.lax.broadcasted_iota(jnp.int32, (1, _NC + 2), 1)
           == _NC + 1).astype(jnp.float32)
    msk_ref[0] = jax.lax.dot_general(
        sel, cls_tar_ref[0], (((1,), (1,)), ((), ())),
        preferred_element_type=jnp.float32, precision=_HI)

    @pl.when(c == _C - 1)
    def _():
        s = acc_s[:, 0:5]
        n = acc_c[:, 0:5]
        mean = s / jnp.maximum(1.0, n)
        mn = jnp.min(mean, axis=1, keepdims=True)
        mx = jnp.max(mean, axis=1, keepdims=True)
        t = 1.0 - (mean - mn) / (mx - mn)
        m1 = jnp.min(t, axis=1, keepdims=True)
        eqm = t == m1
        nmin = jnp.sum(eqm.astype(jnp.float32), axis=1, keepdims=True)
        m2 = jnp.min(jnp.where(eqm, jnp.inf, t), axis=1, keepdims=True)
        mw = jnp.where(nmin >= 2.0, m1, m2)
        tk = jnp.where(t > mw, t, 0.0)
        cnt = cnt_ref[0, 0, 0]
        rowmask = jax.lax.broadcasted_iota(jnp.int32, (8, 5), 0) < cnt
        tk = jnp.where(rowmask, tk, 0.0)
        tstat_ref[0] = jnp.zeros((8, 128), jnp.float32)
        tstat_ref[0, :, 0:5] = tk


def _gather_kernel(tstat_ref, ind_ref, msk_ref, out_ref):
    c = pl.program_id(1)
    validT, lmasks = _lane_masks(c)
    del validT
    oidoh = _oid_onehot(ind_ref[0])
    lvloh = jnp.concatenate(lmasks, axis=0)
    t = tstat_ref[0][:, 0:5]
    w = jax.lax.dot_general(t, lvloh, (((1,), (0,)), ((), ())),
                            preferred_element_type=jnp.float32,
                            precision=_HI)
    gathered = jnp.sum(oidoh * w, axis=0, keepdims=True)
    out_ref[0] = jnp.where(msk_ref[0] > 0.0, gathered, 1.0)


@functools.partial(jax.jit, static_argnums=())
def kernel(cls_pred, loc_pred, cls_tar, loc_tar, ind_tar, bboxes_cnt):
    indT = ind_tar.astype(jnp.int32).reshape(_B, 1, _A)
    cnt32 = bboxes_cnt.astype(jnp.int32).reshape(_B, 1, 1)
    locp_t = jnp.transpose(loc_pred, (0, 2, 1))
    loct_t = jnp.transpose(loc_tar, (0, 2, 1))

    tstat, mskv = pl.pallas_call(
        _stats_kernel,
        grid=(_B, _C),
        in_specs=[
            pl.BlockSpec((1, _TA, _NC), lambda b, c: (b, c, 0)),
            pl.BlockSpec((1, _TA, _NC + 2), lambda b, c: (b, c, 0)),
            pl.BlockSpec((1, 4, _TA), lambda b, c: (b, 0, c)),
            pl.BlockSpec((1, 4, _TA), lambda b, c: (b, 0, c)),
            pl.BlockSpec((1, 1, _TA), lambda b, c: (b, 0, c)),
            pl.BlockSpec((1, 1, 1), lambda b, c: (b, 0, 0),
                         memory_space=pltpu.SMEM),
        ],
        out_specs=[
            pl.BlockSpec((1, 8, 128), lambda b, c: (b, 0, 0)),
            pl.BlockSpec((1, 1, _TA), lambda b, c: (b, 0, c)),
        ],
        out_shape=[
            jax.ShapeDtypeStruct((_B, 8, 128), jnp.float32),
            jax.ShapeDtypeStruct((_B, 1, _A), jnp.float32),
        ],
        scratch_shapes=[pltpu.VMEM((8, 128), jnp.float32),
                        pltpu.VMEM((8, 128), jnp.float32)],
    )(cls_pred, cls_tar, locp_t, loct_t, indT, cnt32)

    outT = pl.pallas_call(
        _gather_kernel,
        grid=(_B, _C),
        in_specs=[
            pl.BlockSpec((1, 8, 128), lambda b, c: (b, 0, 0)),
            pl.BlockSpec((1, 1, _TA), lambda b, c: (b, 0, c)),
            pl.BlockSpec((1, 1, _TA), lambda b, c: (b, 0, c)),
        ],
        out_specs=pl.BlockSpec((1, 1, _TA), lambda b, c: (b, 0, c)),
        out_shape=jax.ShapeDtypeStruct((_B, 1, _A), jnp.float32),
    )(tstat, indT, mskv)

    return outT.reshape(_B, _A, 1)
